# baseline (speedup 1.0000x reference)
import functools

import jax
import jax.numpy as jnp
from jax import lax
from jax.experimental import pallas as pl
from jax.experimental.pallas import tpu as pltpu
from jax.experimental.pallas import tpu_sc as plsc

N = 10000
D = 256
DH = 128
E = 160000
NT = 16
EP = E // NT
K = 50
NCH = EP // K
CB = 20
NG = NCH // CB
NBUF = 4
NRP = 10240
RPT = NRP // NT
NPAD = 10240
CNTW = NPAD // NT
R = 1000
GRID = N // R

_f32 = jnp.float32


@functools.cache
def _make_sc_counts():
    mesh = plsc.VectorSubcoreMesh(core_axis_name="c", subcore_axis_name="s")
    out_type = [jax.ShapeDtypeStruct((6 * NPAD,), _f32)]
    scratch_types = [
        pltpu.VMEM((EP // 16, 16), jnp.int32),
        pltpu.VMEM((NPAD,), _f32),
        pltpu.VMEM((CNTW,), _f32),
        pltpu.VMEM((CNTW,), _f32),
        pltpu.VMEM_SHARED((NT, NPAD), _f32),
    ]

    def body(dsts, z1d, out_c, dst_v, cnt_v, part_v, red_v, cnt_parts):
        c = lax.axis_index("c")
        t = lax.axis_index("s")
        ones = jnp.full((16,), 1.0, _f32)
        for cs in range(2):
            @pl.when(c == cs)
            def _(cs=cs):
                for li in range(3):
                    ell = 3 * cs + li
                    pltpu.sync_copy(dsts.at[ell * NT + t], dst_v)
                    pltpu.sync_copy(z1d, cnt_v)

                    @pl.loop(0, EP // 16)
                    def _(i):
                        idx16 = dst_v[i, pl.ds(0, 16)]
                        plsc.addupdate_scatter(cnt_v, [idx16], ones)

                    pltpu.sync_copy(cnt_v, cnt_parts.at[t])
                    plsc.subcore_barrier()
                    pltpu.sync_copy(z1d.at[pl.ds(0, CNTW)], red_v)

                    @pl.loop(0, NT)
                    def _(p):
                        pltpu.sync_copy(
                            cnt_parts.at[p, pl.ds(t * CNTW, CNTW)], part_v)
                        for k in range(CNTW // 16):
                            sl = pl.ds(k * 16, 16)
                            red_v[sl] += part_v[sl]

                    pltpu.sync_copy(
                        red_v,
                        out_c.at[pl.ds(ell * NPAD + t * CNTW, CNTW)])
                    plsc.subcore_barrier()

    return pl.kernel(body, out_type=out_type, mesh=mesh,
                     scratch_types=scratch_types,
                     compiler_params=pltpu.CompilerParams(
                         needs_layout_passes=False))


@functools.cache
def _make_sc_agg(J):
    mesh = plsc.VectorSubcoreMesh(core_axis_name="c", subcore_axis_name="s")
    out_type = [jax.ShapeDtypeStruct((J, 2, NRP, DH), _f32)]
    scratch_types = [
        pltpu.VMEM((CB, K), jnp.int32),
        pltpu.VMEM((CB, K), jnp.int32),
    ] + [pltpu.VMEM((K, DH), _f32)] * NBUF + [
        pltpu.VMEM_SHARED((NRP, DH), _f32),
    ] + [pltpu.SemaphoreType.DMA] * NBUF

    def body(*refs):
        tables = refs[0:2 * J]
        srcs = refs[2 * J]
        dsts = refs[2 * J + 1]
        z2d = refs[2 * J + 2]
        out_s = refs[2 * J + 3]
        rest = refs[2 * J + 4:]
        src_idx, dst_idx = rest[0], rest[1]
        bufs = rest[2:2 + NBUF]
        acc = rest[2 + NBUF]
        sems = rest[3 + NBUF:3 + 2 * NBUF]

        c = lax.axis_index("c")
        t = lax.axis_index("s")

        for j in range(J):
            for kk in range(RPT // 128):
                pltpu.sync_copy(z2d, acc.at[pl.ds(t * RPT + kk * 128, 128)])
            plsc.subcore_barrier()

            for cs in range(2):
                table = tables[2 * j + cs]

                @pl.when(c == cs)
                def _(table=table):
                    @pl.loop(0, NG)
                    def _(g):
                        grp = (j * NT + t) * NG + g
                        pltpu.sync_copy(srcs.at[grp], src_idx)
                        pltpu.sync_copy(dsts.at[grp], dst_idx)
                        for q in range(NBUF):
                            pltpu.async_copy(
                                table.at[src_idx.at[q]], bufs[q], sems[q])

                        @pl.loop(0, CB // NBUF)
                        def _(hh):
                            base = hh * NBUF
                            for q in range(NBUF):
                                ch = base + q
                                pltpu.make_async_copy(
                                    table.at[src_idx.at[0]], bufs[q],
                                    sems[q]).wait()
                                pltpu.sync_copy(
                                    bufs[q], acc.at[dst_idx.at[ch]], add=True)

                                @pl.when(ch + NBUF < CB)
                                def _(q=q, ch=ch):
                                    pltpu.async_copy(
                                        table.at[src_idx.at[ch + NBUF]],
                                        bufs[q], sems[q])

                                @pl.when(ch + NBUF >= CB)
                                def _(q=q):
                                    pltpu.async_copy(
                                        table.at[src_idx.at[0, pl.ds(0, 1)]],
                                        bufs[q].at[pl.ds(0, 1)], sems[q])

                        for q in range(NBUF):
                            pltpu.make_async_copy(
                                table.at[src_idx.at[0, pl.ds(0, 1)]],
                                bufs[q].at[pl.ds(0, 1)], sems[q]).wait()
            plsc.subcore_barrier()

            for cs in range(2):
                @pl.when(c == cs)
                def _(cs=cs):
                    pltpu.sync_copy(acc.at[pl.ds(t * RPT, RPT)],
                                    out_s.at[j, cs, pl.ds(t * RPT, RPT)])

    return pl.kernel(body, out_type=out_type, mesh=mesh,
                     scratch_types=scratch_types,
                     compiler_params=pltpu.CompilerParams(
                         needs_layout_passes=False))


def _sc_counts(*args):
    return _make_sc_counts()(*args)


def _sc_agg4(*args):
    return _make_sc_agg(4)(*args)


def _sc_agg2(*args):
    return _make_sc_agg(2)(*args)


def _mm(x, w):
    return lax.dot_general(x.astype(jnp.bfloat16), w.astype(jnp.bfloat16),
                           (((1,), (1,)), ((), ())),
                           preferred_element_type=_f32)


def _tc1_body(xA, xP, xS, W, *outs):
    xs = (xA[...], xP[...], xS[...], xP[...])
    combos = ((0, 0, 0), (1, 0, 1), (2, 1, 0), (3, 1, 1))
    w = W[...]
    for j, m, i in combos:
        z = _mm(xs[j], w[m, i])
        outs[2 * j][...] = z[:, :DH]
        outs[2 * j + 1][...] = z[:, DH:]


def _tc1(x_A, x_P, x_S, W_hops):
    blk = pl.BlockSpec((R, D), lambda i: (i, 0))
    half = pl.BlockSpec((R, DH), lambda i: (i, 0))
    return pl.pallas_call(
        _tc1_body,
        grid=(GRID,),
        in_specs=[blk, blk, blk,
                  pl.BlockSpec((2, 2, D, D), lambda i: (0, 0, 0, 0))],
        out_specs=[half] * 8,
        out_shape=[jax.ShapeDtypeStruct((N, DH), _f32)] * 8,
    )(x_A, x_P, x_S, W_hops)


def _tc2_body(s, cnt, xP, bh, Wa, aa, Wl1, Wr1,
              xc0o, xc1o, sb, u00, u01, u10, u11, r0o, r1o):
    i = pl.program_id(0)
    xPb = xP[...]
    cnt4 = cnt[...]
    cw = jnp.where(cnt4 == 0.0, 1.0, cnt4)
    bhv = bh[...]
    xcs = []
    xmps = []
    for m in range(2):
        hops = []
        for k in range(2):
            j = 2 * m + k
            sj = jnp.concatenate([s[j, 0], s[j, 1]], axis=1)
            hops.append((sj + bhv[m, k][None, :]) / cw[:, j:j + 1])
        xmp = hops[0] + hops[1] + xPb
        xmps.append(xmp)
        xcs.append(xmp + hops[1] + xPb)
    xc0o[...] = xcs[0]
    xc1o[...] = xcs[1]

    aav = aa[...]
    parts = []
    for m in range(2):
        th = jnp.tanh(_mm(xcs[m], Wa[...]))
        parts.append(jnp.sum(th * aav[m][None, :]))
    row = lax.broadcasted_iota(jnp.int32, (8, 128), 0)
    col = lax.broadcasted_iota(jnp.int32, (8, 128), 1)
    arr = (jnp.where((row == 0) & (col == 0), parts[0], 0.0)
           + jnp.where((row == 1) & (col == 0), parts[1], 0.0))

    @pl.when(i == 0)
    def _():
        sb[...] = arr

    @pl.when(i > 0)
    def _():
        sb[...] += arr

    wl = Wl1[...]
    wr = Wr1[...]
    us = (u00, u01, u10, u11)
    rs = (r0o, r1o)
    for m in range(2):
        u = _mm(xmps[m], wl)
        us[2 * m][...] = u[:, :DH]
        us[2 * m + 1][...] = u[:, DH:]
        rs[m][...] = _mm(xmps[m], wr)


def _tc2(s_hop, cnt_hop, x_P, b_hops, W_att, a_att, Wl1, Wr1):
    blk = pl.BlockSpec((R, D), lambda i: (i, 0))
    half = pl.BlockSpec((R, DH), lambda i: (i, 0))
    return pl.pallas_call(
        _tc2_body,
        grid=(GRID,),
        in_specs=[
            pl.BlockSpec((4, 2, R, DH), lambda i: (0, 0, i, 0)),
            pl.BlockSpec((R, 4), lambda i: (i, 0)),
            blk,
            pl.BlockSpec((2, 2, D), lambda i: (0, 0, 0)),
            pl.BlockSpec((D, D), lambda i: (0, 0)),
            pl.BlockSpec((2, D), lambda i: (0, 0)),
            pl.BlockSpec((D, D), lambda i: (0, 0)),
            pl.BlockSpec((D, D), lambda i: (0, 0)),
        ],
        out_specs=[blk, blk, pl.BlockSpec((8, 128), lambda i: (0, 0)),
                   half, half, half, half, blk, blk],
        out_shape=[jax.ShapeDtypeStruct((N, D), _f32)] * 2
        + [jax.ShapeDtypeStruct((8, 128), _f32)]
        + [jax.ShapeDtypeStruct((N, DH), _f32)] * 4
        + [jax.ShapeDtypeStruct((N, D), _f32)] * 2,
        compiler_params=pltpu.CompilerParams(
            dimension_semantics=("arbitrary",)),
    )(s_hop, cnt_hop, x_P, b_hops, W_att, a_att, Wl1, Wr1)


def _tc3_body(sb, xc0, xc1, s1, cnte, r0, r1, bl1, Wl2, Wr2,
              xf, u00, u01, u10, u11, r20, r21):
    sbv = sb[...]
    s0 = sbv[0, 0] * (1.0 / N)
    s1v = sbv[1, 0] * (1.0 / N)
    mx = jnp.maximum(s0, s1v)
    e0 = jnp.exp(s0 - mx)
    e1 = jnp.exp(s1v - mx)
    a0 = e0 / (e0 + e1)
    a1 = e1 / (e0 + e1)
    xf[...] = a0 * xc0[...] + a1 * xc1[...]

    cnt2 = jnp.maximum(cnte[...], 1.0)
    blv = bl1[...]
    wl = Wl2[...]
    wr = Wr2[...]
    us = (u00, u01, u10, u11)
    rs = (r20, r21)
    rin = (r0, r1)
    for m in range(2):
        sm = jnp.concatenate([s1[m, 0], s1[m, 1]], axis=1)
        h = jnp.maximum(sm / cnt2[:, m:m + 1] + blv + rin[m][...], 0.0)
        u = _mm(h, wl)
        us[2 * m][...] = u[:, :DH]
        us[2 * m + 1][...] = u[:, DH:]
        rs[m][...] = _mm(h, wr)


def _tc3(sb, xc0, xc1, s1, cnt_enc, r0, r1, bl1, Wl2, Wr2):
    blk = pl.BlockSpec((R, D), lambda i: (i, 0))
    half = pl.BlockSpec((R, DH), lambda i: (i, 0))
    return pl.pallas_call(
        _tc3_body,
        grid=(GRID,),
        in_specs=[
            pl.BlockSpec((8, 128), lambda i: (0, 0)),
            blk, blk,
            pl.BlockSpec((2, 2, R, DH), lambda i: (0, 0, i, 0)),
            pl.BlockSpec((R, 2), lambda i: (i, 0)),
            blk, blk,
            pl.BlockSpec((1, D), lambda i: (0, 0)),
            pl.BlockSpec((D, D), lambda i: (0, 0)),
            pl.BlockSpec((D, D), lambda i: (0, 0)),
        ],
        out_specs=[blk, half, half, half, half, blk, blk],
        out_shape=[jax.ShapeDtypeStruct((N, D), _f32)]
        + [jax.ShapeDtypeStruct((N, DH), _f32)] * 4
        + [jax.ShapeDtypeStruct((N, D), _f32)] * 2,
    )(sb, xc0, xc1, s1, cnt_enc, r0, r1, bl1, Wl2, Wr2)


def _tc4_body(s2, cnte, r20, r21, bl2, o0, o1):
    cnt2 = jnp.maximum(cnte[...], 1.0)
    blv = bl2[...]
    outs = (o0, o1)
    rin = (r20, r21)
    for m in range(2):
        sm = jnp.concatenate([s2[m, 0], s2[m, 1]], axis=1)
        outs[m][...] = sm / cnt2[:, m:m + 1] + blv + rin[m][...]


def _tc4(s2, cnt_enc, r20, r21, bl2):
    blk = pl.BlockSpec((R, D), lambda i: (i, 0))
    return pl.pallas_call(
        _tc4_body,
        grid=(GRID,),
        in_specs=[
            pl.BlockSpec((2, 2, R, DH), lambda i: (0, 0, i, 0)),
            pl.BlockSpec((R, 2), lambda i: (i, 0)),
            blk, blk,
            pl.BlockSpec((1, D), lambda i: (0, 0)),
        ],
        out_specs=[blk, blk],
        out_shape=[jax.ShapeDtypeStruct((N, D), _f32)] * 2,
    )(s2, cnt_enc, r20, r21, bl2)


def kernel(x_P, x_A, x_S, hop_edges, enc_edges, W_hops, b_hops, W_att, a_att,
           Wl1, bl1, Wr1, Wl2, bl2, Wr2):
    hop_src = hop_edges[:, :, 0, :].reshape(4 * NT * NG, CB, K)
    hop_dst_flat = hop_edges[:, :, 1, :].reshape(4, E)
    enc_dst_flat = enc_edges[:, 1, :]
    dsts_all = jnp.concatenate([hop_dst_flat, enc_dst_flat], 0)
    dsts_all = dsts_all.reshape(6 * NT, EP // 16, 16)
    hop_dst = hop_dst_flat.reshape(4 * NT * NG, CB, K)
    enc_src = enc_edges[:, 0, :].reshape(2 * NT * NG, CB, K)
    enc_dst = enc_dst_flat.reshape(2 * NT * NG, CB, K)
    z2d = jnp.zeros((128, DH), _f32)
    z1d = jnp.zeros((NPAD,), _f32)
    bl1_2 = bl1.reshape(1, D)
    bl2_2 = bl2.reshape(1, D)

    (cnt_all_f,) = _sc_counts(dsts_all, z1d)
    cnt_all = cnt_all_f.reshape(6, NPAD)[:, :N]
    cnt_hop = cnt_all[:4].T
    cnt_enc = cnt_all[4:].T

    z = _tc1(x_A, x_P, x_S, W_hops)

    (s_hop,) = _sc_agg4(*z, hop_src, hop_dst, z2d)

    (xc0, xc1, sb, u00, u01, u10, u11, r0, r1) = _tc2(
        s_hop, cnt_hop, x_P, b_hops, W_att, a_att, Wl1, Wr1)

    (s1,) = _sc_agg2(u00, u01, u10, u11, enc_src, enc_dst, z2d)

    (x_fused, v00, v01, v10, v11, r20, r21) = _tc3(
        sb, xc0, xc1, s1, cnt_enc, r0, r1, bl1_2, Wl2, Wr2)

    (s2,) = _sc_agg2(v00, v01, v10, v11, enc_src, enc_dst, z2d)

    enc0, enc1 = _tc4(s2, cnt_enc, r20, r21, bl2_2)
    return x_fused, enc0, enc1


# speedup vs baseline: 1.0341x; 1.0341x over previous
import functools

import jax
import jax.numpy as jnp
from jax import lax
from jax.experimental import pallas as pl
from jax.experimental.pallas import tpu as pltpu
from jax.experimental.pallas import tpu_sc as plsc

N = 10000
D = 256
DH = 128
E = 160000
NT = 16
EP = E // NT
K = 125
NCH = EP // K
CB = 16
NG = NCH // CB
NBUF = 2
NRP = 10240
RPT = NRP // NT
NPAD = 10240
CNTW = NPAD // NT
R = 1000
GRID = N // R

_f32 = jnp.float32


@functools.cache
def _make_sc_counts():
    mesh = plsc.VectorSubcoreMesh(core_axis_name="c", subcore_axis_name="s")
    out_type = [jax.ShapeDtypeStruct((6 * NPAD,), _f32)]
    scratch_types = [
        pltpu.VMEM((EP // 16, 16), jnp.int32),
        pltpu.VMEM((NPAD,), _f32),
        pltpu.VMEM((CNTW,), _f32),
        pltpu.VMEM((CNTW,), _f32),
        pltpu.VMEM_SHARED((NT, NPAD), _f32),
    ]
    lists = [(0, 1), (0, 3), (0, 5), (0, 7), (1, 1), (1, 3)]

    def body(hop8, enc4, z1d, out_c, dst_v, cnt_v, part_v, red_v, cnt_parts):
        c = lax.axis_index("c")
        t = lax.axis_index("s")
        args = (hop8, enc4)
        ones = jnp.full((16,), 1.0, _f32)
        for cs in range(2):
            @pl.when(c == cs)
            def _(cs=cs):
                for li in range(3):
                    ell = 3 * cs + li
                    ai, slab = lists[ell]
                    pltpu.sync_copy(args[ai].at[slab, t], dst_v)
                    pltpu.sync_copy(z1d, cnt_v)

                    @pl.loop(0, EP // 16)
                    def _(i):
                        idx16 = dst_v[i, pl.ds(0, 16)]
                        plsc.addupdate_scatter(cnt_v, [idx16], ones)

                    pltpu.sync_copy(cnt_v, cnt_parts.at[t])
                    plsc.subcore_barrier()
                    pltpu.sync_copy(z1d.at[pl.ds(0, CNTW)], red_v)

                    @pl.loop(0, NT)
                    def _(p):
                        pltpu.sync_copy(
                            cnt_parts.at[p, pl.ds(t * CNTW, CNTW)], part_v)
                        for k in range(CNTW // 16):
                            sl = pl.ds(k * 16, 16)
                            red_v[sl] += part_v[sl]

                    pltpu.sync_copy(
                        red_v,
                        out_c.at[pl.ds(ell * NPAD + t * CNTW, CNTW)])
                    plsc.subcore_barrier()

    return pl.kernel(body, out_type=out_type, mesh=mesh,
                     scratch_types=scratch_types,
                     compiler_params=pltpu.CompilerParams(
                         needs_layout_passes=False))


@functools.cache
def _make_sc_agg(slabs):
    J = len(slabs)
    mesh = plsc.VectorSubcoreMesh(core_axis_name="c", subcore_axis_name="s")
    out_type = [jax.ShapeDtypeStruct((J, 2, NRP, DH), _f32)]
    scratch_types = [
        pltpu.VMEM((CB, K), jnp.int32),
        pltpu.VMEM((CB, K), jnp.int32),
    ] + [pltpu.VMEM((K, DH), _f32)] * NBUF + [
        pltpu.VMEM_SHARED((NRP, DH), _f32),
    ] + [pltpu.SemaphoreType.DMA] * NBUF

    def body(*refs):
        tables = refs[0:2 * J]
        edges = refs[2 * J]
        z2d = refs[2 * J + 1]
        out_s = refs[2 * J + 2]
        rest = refs[2 * J + 3:]
        src_idx, dst_idx = rest[0], rest[1]
        bufs = rest[2:2 + NBUF]
        acc = rest[2 + NBUF]
        sems = rest[3 + NBUF:3 + 2 * NBUF]

        c = lax.axis_index("c")
        t = lax.axis_index("s")

        for j, (s_slab, d_slab) in enumerate(slabs):
            for kk in range(RPT // 128):
                pltpu.sync_copy(z2d, acc.at[pl.ds(t * RPT + kk * 128, 128)])
            plsc.subcore_barrier()

            for cs in range(2):
                table = tables[2 * j + cs]

                @pl.when(c == cs)
                def _(table=table, s_slab=s_slab, d_slab=d_slab):
                    @pl.loop(0, NG)
                    def _(g):
                        pltpu.sync_copy(
                            edges.at[(s_slab * NT + t) * NG + g], src_idx)
                        pltpu.sync_copy(
                            edges.at[(d_slab * NT + t) * NG + g], dst_idx)
                        for q in range(NBUF):
                            pltpu.async_copy(
                                table.at[src_idx.at[q]], bufs[q], sems[q])

                        @pl.loop(0, CB // NBUF)
                        def _(hh):
                            base = hh * NBUF
                            for q in range(NBUF):
                                ch = base + q
                                pltpu.make_async_copy(
                                    table.at[src_idx.at[0]], bufs[q],
                                    sems[q]).wait()
                                pltpu.sync_copy(
                                    bufs[q], acc.at[dst_idx.at[ch]], add=True)

                                @pl.when(ch + NBUF < CB)
                                def _(q=q, ch=ch):
                                    pltpu.async_copy(
                                        table.at[src_idx.at[ch + NBUF]],
                                        bufs[q], sems[q])

                                @pl.when(ch + NBUF >= CB)
                                def _(q=q):
                                    pltpu.async_copy(
                                        table.at[src_idx.at[0, pl.ds(0, 1)]],
                                        bufs[q].at[pl.ds(0, 1)], sems[q])

                        for q in range(NBUF):
                            pltpu.make_async_copy(
                                table.at[src_idx.at[0, pl.ds(0, 1)]],
                                bufs[q].at[pl.ds(0, 1)], sems[q]).wait()
            plsc.subcore_barrier()

            for cs in range(2):
                @pl.when(c == cs)
                def _(cs=cs, j=j):
                    pltpu.sync_copy(acc.at[pl.ds(t * RPT, RPT)],
                                    out_s.at[j, cs, pl.ds(t * RPT, RPT)])

    return pl.kernel(body, out_type=out_type, mesh=mesh,
                     scratch_types=scratch_types,
                     compiler_params=pltpu.CompilerParams(
                         needs_layout_passes=False))


def _sc_counts(*args):
    return _make_sc_counts()(*args)


def _sc_agg4(*args):
    return _make_sc_agg(((0, 1), (2, 3), (4, 5), (6, 7)))(*args)


def _sc_agg2(*args):
    return _make_sc_agg(((0, 1), (2, 3)))(*args)


def _mm(x, w):
    return lax.dot_general(x.astype(jnp.bfloat16), w.astype(jnp.bfloat16),
                           (((1,), (1,)), ((), ())),
                           preferred_element_type=_f32)


_BLK = pl.BlockSpec((R, D), lambda i: (i, 0))
_HALF = pl.BlockSpec((R, DH), lambda i: (i, 0))
_CNT = pl.BlockSpec((R, 6), lambda i: (i, 0))
_W = pl.BlockSpec((D, D), lambda i: (0, 0))


def _tc1_body(xA, xP, xS, W, *outs):
    xs = (xA[...], xP[...], xS[...], xP[...])
    combos = ((0, 0, 0), (1, 0, 1), (2, 1, 0), (3, 1, 1))
    w = W[...]
    for j, m, i in combos:
        z = _mm(xs[j], w[m, i])
        outs[2 * j][...] = z[:, :DH]
        outs[2 * j + 1][...] = z[:, DH:]


def _tc1(x_A, x_P, x_S, W_hops):
    return pl.pallas_call(
        _tc1_body,
        grid=(GRID,),
        in_specs=[_BLK, _BLK, _BLK,
                  pl.BlockSpec((2, 2, D, D), lambda i: (0, 0, 0, 0))],
        out_specs=[_HALF] * 8,
        out_shape=[jax.ShapeDtypeStruct((N, DH), _f32)] * 8,
    )(x_A, x_P, x_S, W_hops)


def _tc2a_body(s, cnt, xP, bh, Wl1,
               u00, u01, u10, u11, xm0o, xm1o, xc0o, xc1o):
    xPb = xP[...]
    cnt6 = cnt[...]
    bhv = bh[...]
    wl = Wl1[...]
    us = (u00, u01, u10, u11)
    xmo = (xm0o, xm1o)
    xco = (xc0o, xc1o)
    for m in range(2):
        hops = []
        for k in range(2):
            j = 2 * m + k
            sj = jnp.concatenate([s[j, 0], s[j, 1]], axis=1)
            cj = cnt6[:, j:j + 1]
            cw = jnp.where(cj == 0.0, 1.0, cj)
            hops.append((sj + bhv[m, k][None, :]) / cw)
        xmp = hops[0] + hops[1] + xPb
        xmo[m][...] = xmp
        xco[m][...] = xmp + hops[1] + xPb
        u = _mm(xmp, wl)
        us[2 * m][...] = u[:, :DH]
        us[2 * m + 1][...] = u[:, DH:]


def _tc2a(s_hop, cnt_t, x_P, b_hops, Wl1):
    return pl.pallas_call(
        _tc2a_body,
        grid=(GRID,),
        in_specs=[
            pl.BlockSpec((4, 2, R, DH), lambda i: (0, 0, i, 0)),
            _CNT, _BLK,
            pl.BlockSpec((2, 2, D), lambda i: (0, 0, 0)),
            _W,
        ],
        out_specs=[_HALF] * 4 + [_BLK] * 4,
        out_shape=[jax.ShapeDtypeStruct((N, DH), _f32)] * 4
        + [jax.ShapeDtypeStruct((N, D), _f32)] * 4,
    )(s_hop, cnt_t, x_P, b_hops, Wl1)


def _tc2b_body(xm0, xm1, xc0, xc1, Wa, aa, Wr1, sb, r0o, r1o):
    i = pl.program_id(0)
    aav = aa[...]
    parts = []
    for m, xc in enumerate((xc0, xc1)):
        th = jnp.tanh(_mm(xc[...], Wa[...]))
        parts.append(jnp.sum(th * aav[m][None, :]))
    row = lax.broadcasted_iota(jnp.int32, (8, 128), 0)
    col = lax.broadcasted_iota(jnp.int32, (8, 128), 1)
    arr = (jnp.where((row == 0) & (col == 0), parts[0], 0.0)
           + jnp.where((row == 1) & (col == 0), parts[1], 0.0))

    @pl.when(i == 0)
    def _():
        sb[...] = arr

    @pl.when(i > 0)
    def _():
        sb[...] += arr

    wr = Wr1[...]
    r0o[...] = _mm(xm0[...], wr)
    r1o[...] = _mm(xm1[...], wr)


def _tc2b(xm0, xm1, xc0, xc1, W_att, a_att, Wr1):
    return pl.pallas_call(
        _tc2b_body,
        grid=(GRID,),
        in_specs=[_BLK, _BLK, _BLK, _BLK, _W,
                  pl.BlockSpec((2, D), lambda i: (0, 0)), _W],
        out_specs=[pl.BlockSpec((8, 128), lambda i: (0, 0)), _BLK, _BLK],
        out_shape=[jax.ShapeDtypeStruct((8, 128), _f32)]
        + [jax.ShapeDtypeStruct((N, D), _f32)] * 2,
        compiler_params=pltpu.CompilerParams(
            dimension_semantics=("arbitrary",)),
    )(xm0, xm1, xc0, xc1, W_att, a_att, Wr1)


def _tc3a_body(s1, cnt, r0, r1, bl1, Wl2,
               h0o, h1o, u00, u01, u10, u11):
    cnt6 = cnt[...]
    blv = bl1[...]
    wl = Wl2[...]
    us = (u00, u01, u10, u11)
    hs = (h0o, h1o)
    rin = (r0, r1)
    for m in range(2):
        sm = jnp.concatenate([s1[m, 0], s1[m, 1]], axis=1)
        cw = jnp.maximum(cnt6[:, 4 + m:5 + m], 1.0)
        h = jnp.maximum(sm / cw + blv + rin[m][...], 0.0)
        hs[m][...] = h
        u = _mm(h, wl)
        us[2 * m][...] = u[:, :DH]
        us[2 * m + 1][...] = u[:, DH:]


def _tc3a(s1, cnt_t, r0, r1, bl1, Wl2):
    return pl.pallas_call(
        _tc3a_body,
        grid=(GRID,),
        in_specs=[
            pl.BlockSpec((2, 2, R, DH), lambda i: (0, 0, i, 0)),
            _CNT, _BLK, _BLK,
            pl.BlockSpec((1, D), lambda i: (0, 0)),
            _W,
        ],
        out_specs=[_BLK, _BLK] + [_HALF] * 4,
        out_shape=[jax.ShapeDtypeStruct((N, D), _f32)] * 2
        + [jax.ShapeDtypeStruct((N, DH), _f32)] * 4,
    )(s1, cnt_t, r0, r1, bl1, Wl2)


def _tc3b_body(sb, xc0, xc1, h0, h1, Wr2, xf, r20, r21):
    sbv = sb[...]
    s0 = sbv[0, 0] * (1.0 / N)
    s1v = sbv[1, 0] * (1.0 / N)
    mx = jnp.maximum(s0, s1v)
    e0 = jnp.exp(s0 - mx)
    e1 = jnp.exp(s1v - mx)
    a0 = e0 / (e0 + e1)
    a1 = e1 / (e0 + e1)
    xf[...] = a0 * xc0[...] + a1 * xc1[...]
    wr = Wr2[...]
    r20[...] = _mm(h0[...], wr)
    r21[...] = _mm(h1[...], wr)


def _tc3b(sb, xc0, xc1, h0, h1, Wr2):
    return pl.pallas_call(
        _tc3b_body,
        grid=(GRID,),
        in_specs=[pl.BlockSpec((8, 128), lambda i: (0, 0)),
                  _BLK, _BLK, _BLK, _BLK, _W],
        out_specs=[_BLK, _BLK, _BLK],
        out_shape=[jax.ShapeDtypeStruct((N, D), _f32)] * 3,
    )(sb, xc0, xc1, h0, h1, Wr2)


def _tc4_body(s2, cnt, r20, r21, bl2, o0, o1):
    cnt6 = cnt[...]
    blv = bl2[...]
    outs = (o0, o1)
    rin = (r20, r21)
    for m in range(2):
        sm = jnp.concatenate([s2[m, 0], s2[m, 1]], axis=1)
        cw = jnp.maximum(cnt6[:, 4 + m:5 + m], 1.0)
        outs[m][...] = sm / cw + blv + rin[m][...]


def _tc4(s2, cnt_t, r20, r21, bl2):
    return pl.pallas_call(
        _tc4_body,
        grid=(GRID,),
        in_specs=[
            pl.BlockSpec((2, 2, R, DH), lambda i: (0, 0, i, 0)),
            _CNT, _BLK, _BLK,
            pl.BlockSpec((1, D), lambda i: (0, 0)),
        ],
        out_specs=[_BLK, _BLK],
        out_shape=[jax.ShapeDtypeStruct((N, D), _f32)] * 2,
    )(s2, cnt_t, r20, r21, bl2)


def kernel(x_P, x_A, x_S, hop_edges, enc_edges, W_hops, b_hops, W_att, a_att,
           Wl1, bl1, Wr1, Wl2, bl2, Wr2):
    hop_flat = hop_edges.reshape(8 * NT * NG, CB, K)
    enc_flat = enc_edges.reshape(4 * NT * NG, CB, K)
    hop_cnt = hop_edges.reshape(8, NT, EP // 16, 16)
    enc_cnt = enc_edges.reshape(4, NT, EP // 16, 16)
    z2d = jnp.zeros((128, DH), _f32)
    z1d = jnp.zeros((NPAD,), _f32)
    bl1_2 = bl1.reshape(1, D)
    bl2_2 = bl2.reshape(1, D)

    (cnt_f,) = _sc_counts(hop_cnt, enc_cnt, z1d)
    cnt_t = cnt_f.reshape(6, NPAD).T

    z = _tc1(x_A, x_P, x_S, W_hops)

    (s_hop,) = _sc_agg4(*z, hop_flat, z2d)

    (u00, u01, u10, u11, xm0, xm1, xc0, xc1) = _tc2a(
        s_hop, cnt_t, x_P, b_hops, Wl1)

    (s1,) = _sc_agg2(u00, u01, u10, u11, enc_flat, z2d)

    (sb, r0, r1) = _tc2b(xm0, xm1, xc0, xc1, W_att, a_att, Wr1)

    (h0, h1, v00, v01, v10, v11) = _tc3a(s1, cnt_t, r0, r1, bl1_2, Wl2)

    (s2,) = _sc_agg2(v00, v01, v10, v11, enc_flat, z2d)

    (x_fused, r20, r21) = _tc3b(sb, xc0, xc1, h0, h1, Wr2)

    enc0, enc1 = _tc4(s2, cnt_t, r20, r21, bl2_2)
    return x_fused, enc0, enc1
